# baseline (speedup 1.0000x reference)
import dataclasses

import jax
import jax.numpy as jnp
from jax import lax
from jax.experimental import pallas as pl
from jax.experimental.pallas import tpu as pltpu
from jax.experimental.pallas import tpu_sc as plsc

_D = 100
_R = 32
_DP = 128
_BA = 10
_RB = 2000
_W = 128
_L = 16


def _pairs_body(c0_ref, c1f_ref, p_ref):
    ba = p_ref.shape[0]
    c0 = c0_ref[...].reshape(ba * _R, _R)
    p = jnp.dot(c0, c1f_ref[...], preferred_element_type=jnp.float32)
    p = p.astype(jnp.bfloat16).reshape(ba, _R, _D * _R)
    p_ref[...] = p.transpose(0, 2, 1)


def _build_pairs(c0, c1f):
    return pl.pallas_call(
        _pairs_body,
        grid=(_D // _BA,),
        in_specs=[
            pl.BlockSpec((_BA, _R, _R), lambda g: (g, 0, 0)),
            pl.BlockSpec((_R, _D * _R), lambda g: (0, 0)),
        ],
        out_specs=pl.BlockSpec((_BA, _D * _R, _R), lambda g: (g, 0, 0)),
        out_shape=jax.ShapeDtypeStruct((_D, _D * _R, _R), jnp.bfloat16),
    )(c0, c1f)


def _table_body(p2_ref, c2m_ref, t_ref):
    t_ref[...] = jnp.dot(
        p2_ref[...], c2m_ref[...], preferred_element_type=jnp.float32
    )


def _build_table(p2, c2m):
    return pl.pallas_call(
        _table_body,
        grid=(_D * _D // _RB,),
        in_specs=[
            pl.BlockSpec((_RB, _R * _R), lambda g: (g, 0)),
            pl.BlockSpec((_R * _R, _DP), lambda g: (0, 0)),
        ],
        out_specs=pl.BlockSpec((_RB, _DP), lambda g: (g, 0)),
        out_shape=jax.ShapeDtypeStruct((_D * _D, _DP), jnp.float32),
    )(p2, c2m)


def _gather_table(t2, i0, i1, i2):
    b = i0.shape[1]
    mesh = plsc.VectorSubcoreMesh(core_axis_name="c", subcore_axis_name="s")
    cp = pltpu.CompilerParams()
    if "needs_layout_passes" in pltpu.CompilerParams.__dataclass_fields__:
        cp = dataclasses.replace(cp, needs_layout_passes=False)

    @pl.kernel(
        out_type=jax.ShapeDtypeStruct((1, b), jnp.float32),
        mesh=mesh,
        scratch_types=[
            pltpu.VMEM((1, _W), jnp.int32),
            pltpu.VMEM((_W, _DP), jnp.float32),
        ],
        compiler_params=cp,
    )
    def k(t_hbm, i0_hbm, i1_hbm, i2_hbm, o_hbm, flat_ref, rows_ref):
        def body(i0_v, i1_v, i2_v, o_v):
            @pl.loop(0, _W, step=_L)
            def _(c):
                s = (0, pl.ds(c, _L))
                flat_ref.at[*s][...] = i0_v.at[*s][...] * _D + i1_v.at[*s][...]
            pltpu.sync_copy(t_hbm.at[flat_ref.at[0]], rows_ref)

            @pl.loop(0, _W, step=_L)
            def _(c):
                s = (0, pl.ds(c, _L))
                row_ids = lax.iota(jnp.int32, _L) + c
                o_v.at[*s][...] = plsc.load_gather(
                    rows_ref, [row_ids, i2_v.at[*s][...]]
                )

        pltpu.emit_pipeline(
            body,
            grid=(b // _W,),
            in_specs=[pl.BlockSpec((1, _W), lambda i: (0, i))] * 3,
            out_specs=[pl.BlockSpec((1, _W), lambda i: (0, i))],
            core_axis_name=("c", "s"),
            dimension_semantics=(pltpu.PARALLEL,),
        )(i0_hbm, i1_hbm, i2_hbm, o_hbm)

    return k(t2, i0, i1, i2)


def kernel(index, core0, core1, core2):
    c1f = core1.transpose(1, 0, 2).reshape(_R, _D * _R)
    c2m = core2.transpose(1, 2, 0).reshape(_R * _R, _D)
    c2m = jnp.pad(c2m, ((0, 0), (0, _DP - _D)))
    p = _build_pairs(core0.astype(jnp.bfloat16), c1f.astype(jnp.bfloat16))
    p2 = p.reshape(_D * _D, _R * _R)
    t2 = _build_table(p2, c2m.astype(jnp.bfloat16))
    idx = index.astype(jnp.int32)
    i0 = idx[:, 0].reshape(1, -1)
    i1 = idx[:, 1].reshape(1, -1)
    i2 = idx[:, 2].reshape(1, -1)
    out = _gather_table(t2, i0, i1, i2)
    return out.reshape(-1)


# speedup vs baseline: 1.7622x; 1.7622x over previous
import dataclasses

import jax
import jax.numpy as jnp
from jax import lax
from jax.experimental import pallas as pl
from jax.experimental.pallas import tpu as pltpu
from jax.experimental.pallas import tpu_sc as plsc

_D = 100
_R = 32
_DP = 128
_BA = 10
_RB = 2000
_W = 128
_L = 16


def _table_body(c0_ref, c1f_ref, c2m_ref, t_ref):
    ba = c0_ref.shape[0]
    c0 = c0_ref[...].reshape(ba * _R, _R)
    p = jnp.dot(c0, c1f_ref[...], preferred_element_type=jnp.float32)
    p = p.astype(jnp.bfloat16).reshape(ba, _R, _D, _R)
    pr = p.transpose(0, 2, 1, 3).reshape(ba * _D, _R * _R)
    t_ref[...] = jnp.dot(pr, c2m_ref[...], preferred_element_type=jnp.float32)


def _build_table(c0, c1f, c2m):
    return pl.pallas_call(
        _table_body,
        grid=(_D // _BA,),
        in_specs=[
            pl.BlockSpec((_BA, _R, _R), lambda g: (g, 0, 0)),
            pl.BlockSpec((_R, _D * _R), lambda g: (0, 0)),
            pl.BlockSpec((_R * _R, _DP), lambda g: (0, 0)),
        ],
        out_specs=pl.BlockSpec((_BA * _D, _DP), lambda g: (g, 0)),
        out_shape=jax.ShapeDtypeStruct((_D * _D, _DP), jnp.float32),
    )(c0, c1f, c2m)


def _gather_table(t2, i0, i1, i2):
    b = i0.shape[1]
    mesh = plsc.VectorSubcoreMesh(core_axis_name="c", subcore_axis_name="s")
    cp = pltpu.CompilerParams()
    if "needs_layout_passes" in pltpu.CompilerParams.__dataclass_fields__:
        cp = dataclasses.replace(cp, needs_layout_passes=False)

    @pl.kernel(
        out_type=jax.ShapeDtypeStruct((1, b), jnp.float32),
        mesh=mesh,
        scratch_types=[
            pltpu.VMEM((1, _W), jnp.int32),
            pltpu.VMEM((_W, _DP), jnp.float32),
        ],
        compiler_params=cp,
    )
    def k(t_hbm, i0_hbm, i1_hbm, i2_hbm, o_hbm, flat_ref, rows_ref):
        def body(i0_v, i1_v, i2_v, o_v):
            @pl.loop(0, _W, step=_L)
            def _(c):
                s = (0, pl.ds(c, _L))
                flat_ref.at[*s][...] = i0_v.at[*s][...] * _D + i1_v.at[*s][...]
            pltpu.sync_copy(t_hbm.at[flat_ref.at[0]], rows_ref)

            @pl.loop(0, _W, step=_L)
            def _(c):
                s = (0, pl.ds(c, _L))
                row_ids = lax.iota(jnp.int32, _L) + c
                o_v.at[*s][...] = plsc.load_gather(
                    rows_ref, [row_ids, i2_v.at[*s][...]]
                )

        pltpu.emit_pipeline(
            body,
            grid=(b // _W,),
            in_specs=[pl.BlockSpec((1, _W), lambda i: (0, i))] * 3,
            out_specs=[pl.BlockSpec((1, _W), lambda i: (0, i))],
            core_axis_name=("c", "s"),
            dimension_semantics=(pltpu.PARALLEL,),
        )(i0_hbm, i1_hbm, i2_hbm, o_hbm)

    return k(t2, i0, i1, i2)


def kernel(index, core0, core1, core2):
    c1f = core1.transpose(1, 0, 2).reshape(_R, _D * _R)
    c2m = core2.transpose(2, 1, 0).reshape(_R * _R, _D)
    c2m = jnp.pad(c2m, ((0, 0), (0, _DP - _D)))
    t2 = _build_table(
        core0.astype(jnp.bfloat16),
        c1f.astype(jnp.bfloat16),
        c2m.astype(jnp.bfloat16),
    )
    idx = index.astype(jnp.int32)
    i0 = idx[:, 0].reshape(1, -1)
    i1 = idx[:, 1].reshape(1, -1)
    i2 = idx[:, 2].reshape(1, -1)
    out = _gather_table(t2, i0, i1, i2)
    return out.reshape(-1)


# speedup vs baseline: 2.1513x; 1.2208x over previous
import dataclasses

import jax
import jax.numpy as jnp
from jax import lax
from jax.experimental import pallas as pl
from jax.experimental.pallas import tpu as pltpu
from jax.experimental.pallas import tpu_sc as plsc

_D = 100
_R = 32
_DP = 128
_BA = 10
_RB = 2000
_W = 128
_L = 16


def _table_body(c0_ref, c1f_ref, c2m_ref, t_ref):
    ba = c0_ref.shape[0]
    c0 = c0_ref[...].reshape(ba * _R, _R)
    p = jnp.dot(c0, c1f_ref[...], preferred_element_type=jnp.float32)
    p = p.astype(jnp.bfloat16).reshape(ba, _R, _R * _DP)
    pt = p.transpose(0, 2, 1)
    acc = jnp.zeros((ba * _DP, _DP), jnp.float32)
    for o in range(_R // 8):
        lhs = jnp.concatenate(
            [pt[:, (8 * o + m) * _DP:(8 * o + m + 1) * _DP, :] for m in range(8)],
            axis=2,
        ).reshape(ba * _DP, 8 * _R)
        acc = acc + jnp.dot(
            lhs, c2m_ref[8 * _R * o:8 * _R * (o + 1), :],
            preferred_element_type=jnp.float32,
        )
    t_ref[...] = acc


def _build_table(c0, c1f, c2m):
    return pl.pallas_call(
        _table_body,
        grid=(_D // _BA,),
        in_specs=[
            pl.BlockSpec((_BA, _R, _R), lambda g: (g, 0, 0)),
            pl.BlockSpec((_R, _R * _DP), lambda g: (0, 0)),
            pl.BlockSpec((_R * _R, _DP), lambda g: (0, 0)),
        ],
        out_specs=pl.BlockSpec((_BA * _DP, _DP), lambda g: (g, 0)),
        out_shape=jax.ShapeDtypeStruct((_D * _DP, _DP), jnp.float32),
    )(c0, c1f, c2m)


def _gather_table(t2, i0, i1, i2):
    b = i0.shape[1]
    mesh = plsc.VectorSubcoreMesh(core_axis_name="c", subcore_axis_name="s")
    cp = pltpu.CompilerParams()
    if "needs_layout_passes" in pltpu.CompilerParams.__dataclass_fields__:
        cp = dataclasses.replace(cp, needs_layout_passes=False)

    @pl.kernel(
        out_type=jax.ShapeDtypeStruct((1, b), jnp.float32),
        mesh=mesh,
        scratch_types=[
            pltpu.VMEM((1, _W), jnp.int32),
            pltpu.VMEM((_W, _DP), jnp.float32),
        ],
        compiler_params=cp,
    )
    def k(t_hbm, i0_hbm, i1_hbm, i2_hbm, o_hbm, flat_ref, rows_ref):
        def body(i0_v, i1_v, i2_v, o_v):
            @pl.loop(0, _W, step=_L)
            def _(c):
                s = (0, pl.ds(c, _L))
                flat_ref.at[*s][...] = i0_v.at[*s][...] * _DP + i1_v.at[*s][...]
            pltpu.sync_copy(t_hbm.at[flat_ref.at[0]], rows_ref)

            @pl.loop(0, _W, step=_L)
            def _(c):
                s = (0, pl.ds(c, _L))
                row_ids = lax.iota(jnp.int32, _L) + c
                o_v.at[*s][...] = plsc.load_gather(
                    rows_ref, [row_ids, i2_v.at[*s][...]]
                )

        pltpu.emit_pipeline(
            body,
            grid=(b // _W,),
            in_specs=[pl.BlockSpec((1, _W), lambda i: (0, i))] * 3,
            out_specs=[pl.BlockSpec((1, _W), lambda i: (0, i))],
            core_axis_name=("c", "s"),
            dimension_semantics=(pltpu.PARALLEL,),
        )(i0_hbm, i1_hbm, i2_hbm, o_hbm)

    return k(t2, i0, i1, i2)


def kernel(index, core0, core1, core2):
    c1f = jnp.pad(
        core1.transpose(1, 2, 0), ((0, 0), (0, 0), (0, _DP - _D))
    ).reshape(_R, _R * _DP)
    c2m = core2.transpose(1, 2, 0).reshape(_R * _R, _D)
    c2m = jnp.pad(c2m, ((0, 0), (0, _DP - _D)))
    t2 = _build_table(
        core0.astype(jnp.bfloat16),
        c1f.astype(jnp.bfloat16),
        c2m.astype(jnp.bfloat16),
    )
    idx = index.astype(jnp.int32)
    i0 = idx[:, 0].reshape(1, -1)
    i1 = idx[:, 1].reshape(1, -1)
    i2 = idx[:, 2].reshape(1, -1)
    out = _gather_table(t2, i0, i1, i2)
    return out.reshape(-1)
